# baseline (speedup 1.0000x reference)
import functools

import jax
import jax.numpy as jnp
from jax import lax
from jax.experimental import pallas as pl
from jax.experimental.pallas import tpu as pltpu
from jax.experimental.pallas import tpu_sc as plsc

N = 10000
E = 320000
D = 128
L = 3
NC = 2
NS = 16
NW = NC * NS
CH = 128
NCHUNK = 79
PER_TILE = NCHUNK * CH
ROWS_PER_TILE = 632
NP = NS * ROWS_PER_TILE
DEG_W = 128

@functools.lru_cache(maxsize=None)
def _make_sc_agg(width):
    @functools.partial(
        pl.kernel,
        mesh=plsc.VectorSubcoreMesh(core_axis_name="c", subcore_axis_name="s"),
        out_type=jax.ShapeDtypeStruct((NC * NP, width), jnp.float32),
        scratch_types=[
            pltpu.VMEM((NCHUNK, CH), jnp.int32),
            pltpu.VMEM((NCHUNK, CH), jnp.int32),
            pltpu.VMEM((CH, width), jnp.float32),
            pltpu.VMEM_SHARED((NP, width), jnp.float32),
            pltpu.SemaphoreType.DMA,
        ],
    )
    def sc_agg(src_hbm, dst_hbm, table_hbm, zeros_hbm, out_hbm,
               src_v, dst_v, rows_v, acc, sem):
        cid = lax.axis_index("c")
        sid = lax.axis_index("s")
        tid = cid * NS + sid
        r0 = sid * ROWS_PER_TILE
        pltpu.sync_copy(zeros_hbm.at[pl.ds(r0, ROWS_PER_TILE)],
                        acc.at[pl.ds(r0, ROWS_PER_TILE)])
        pltpu.sync_copy(src_hbm.at[tid], src_v)
        pltpu.sync_copy(dst_hbm.at[tid], dst_v)
        plsc.subcore_barrier()

        def body(j, carry):
            pltpu.async_copy(table_hbm.at[src_v.at[j]], rows_v, sem).wait()
            pltpu.sync_copy(rows_v, acc.at[dst_v.at[j]], add=True)
            return carry

        lax.fori_loop(0, NCHUNK, body, 0)
        plsc.subcore_barrier()
        pltpu.sync_copy(acc.at[pl.ds(r0, ROWS_PER_TILE)],
                        out_hbm.at[pl.ds(cid * NP + r0, ROWS_PER_TILE)])

    return sc_agg


@functools.lru_cache(maxsize=None)
def _make_sc_deg():
    @functools.partial(
        pl.kernel,
        mesh=plsc.VectorSubcoreMesh(core_axis_name="c", subcore_axis_name="s"),
        out_type=jax.ShapeDtypeStruct((NC * NP, DEG_W), jnp.float32),
        scratch_types=[
            pltpu.VMEM((NCHUNK, CH), jnp.int32),
            pltpu.VMEM((CH, DEG_W), jnp.float32),
            pltpu.VMEM_SHARED((NP, DEG_W), jnp.float32),
        ],
    )
    def sc_deg(dst_hbm, ones_hbm, zeros_hbm, out_hbm, dst_v, ones_v, acc):
        cid = lax.axis_index("c")
        sid = lax.axis_index("s")
        tid = cid * NS + sid
        r0 = sid * ROWS_PER_TILE
        pltpu.sync_copy(zeros_hbm.at[pl.ds(r0, ROWS_PER_TILE)],
                        acc.at[pl.ds(r0, ROWS_PER_TILE)])
        pltpu.sync_copy(ones_hbm, ones_v)
        pltpu.sync_copy(dst_hbm.at[tid], dst_v)
        plsc.subcore_barrier()

        def body(j, carry):
            pltpu.sync_copy(ones_v, acc.at[dst_v.at[j]], add=True)
            return carry

        lax.fori_loop(0, NCHUNK, body, 0)
        plsc.subcore_barrier()
        pltpu.sync_copy(acc.at[pl.ds(r0, ROWS_PER_TILE)],
                        out_hbm.at[pl.ds(cid * NP + r0, ROWS_PER_TILE)])

    return sc_deg


def _tc_emb_body(h_ref, w_ref, b_ref, o_ref):
    o_ref[...] = jnp.dot(h_ref[...], w_ref[...],
                         preferred_element_type=jnp.float32) + b_ref[...]


def _tc_layer_body(x_ref, s_ref, degp_ref, w_ref, b_ref, g_ref, bet_ref,
                   o_ref):
    x = x_ref[...]
    s = s_ref[0:N] + s_ref[NP:NP + N]
    dp = degp_ref[0:N] + degp_ref[NP:NP + N]
    inv = 1.0 / jnp.maximum(dp, 1.0)
    w = w_ref[...]
    bundle = (jnp.dot(x, w[0:D], preferred_element_type=jnp.float32)
              + jnp.dot(s * inv, w[D:2 * D],
                        preferred_element_type=jnp.float32)
              + b_ref[...])
    nrm = jnp.sqrt(jnp.sum(bundle * bundle, axis=1, keepdims=True))
    bundle = bundle / jnp.maximum(nrm, 1e-12)
    y = jnp.maximum(bundle, 0.0)
    mean = jnp.mean(y, axis=0, keepdims=True)
    var = jnp.mean((y - mean) ** 2, axis=0, keepdims=True)
    o_ref[...] = x + (y - mean) * lax.rsqrt(var + 1e-5) * g_ref[...] \
        + bet_ref[...]


@functools.lru_cache(maxsize=None)
def _make_tc():
    tc_emb = pl.pallas_call(
        _tc_emb_body,
        out_shape=jax.ShapeDtypeStruct((N, D), jnp.float32),
    )
    tc_layer = pl.pallas_call(
        _tc_layer_body,
        out_shape=jax.ShapeDtypeStruct((N, D), jnp.float32),
    )
    return tc_emb, tc_layer


def kernel(h, e, edge_index, W_emb, b_emb, Ws, bs, gammas, betas):
    pad = PER_TILE - E // NW
    src = edge_index[0].astype(jnp.int32).reshape(NW, E // NW)
    dst = edge_index[1].astype(jnp.int32).reshape(NW, E // NW)
    src = jnp.pad(src, ((0, 0), (0, pad))).reshape(NW, NCHUNK, CH)
    dst = jnp.pad(dst, ((0, 0), (0, pad)),
                  constant_values=N).reshape(NW, NCHUNK, CH)
    zeros_g = jnp.zeros((NP, DEG_W), jnp.float32)
    ones_g = jnp.ones((CH, DEG_W), jnp.float32)
    sc_agg_rows = _make_sc_agg(D)
    sc_deg = _make_sc_deg()
    tc_emb, tc_layer = _make_tc()
    degp = sc_deg(dst, ones_g, zeros_g)
    x = tc_emb(h, W_emb, b_emb[None, :])
    zeros_d = degp[0:NP] * 0.0
    for l in range(L):
        s = sc_agg_rows(src, dst, x, zeros_d)
        x = tc_layer(x, s, degp, Ws[l], bs[l][None, :],
                     gammas[l][None, :], betas[l][None, :])
    return x


# speedup vs baseline: 1.3741x; 1.3741x over previous
import functools

import jax
import jax.numpy as jnp
from jax import lax
from jax.experimental import pallas as pl
from jax.experimental.pallas import tpu as pltpu
from jax.experimental.pallas import tpu_sc as plsc

N = 10000
E = 320000
D = 128
L = 3
NC = 2
NS = 16
NW = NC * NS
CH = 80
NCHUNK = 125
PER_TILE = NCHUNK * CH
ROWS_PER_TILE = 632
NP = NS * ROWS_PER_TILE
DEG_W = 128

@functools.lru_cache(maxsize=None)
def _make_sc_agg(width):
    @functools.partial(
        pl.kernel,
        mesh=plsc.VectorSubcoreMesh(core_axis_name="c", subcore_axis_name="s"),
        out_type=jax.ShapeDtypeStruct((NC * NP, width), jnp.float32),
        scratch_types=[
            pltpu.VMEM((NCHUNK, CH), jnp.int32),
            pltpu.VMEM((NCHUNK, CH), jnp.int32),
            pltpu.VMEM((CH, width), jnp.float32),
            pltpu.VMEM_SHARED((NP, width), jnp.float32),
            pltpu.SemaphoreType.DMA,
        ],
    )
    def sc_agg(src_hbm, dst_hbm, table_hbm, zeros_hbm, out_hbm,
               src_v, dst_v, rows_v, acc, sem):
        cid = lax.axis_index("c")
        sid = lax.axis_index("s")
        tid = cid * NS + sid
        r0 = sid * ROWS_PER_TILE
        pltpu.sync_copy(zeros_hbm.at[pl.ds(r0, ROWS_PER_TILE)],
                        acc.at[pl.ds(r0, ROWS_PER_TILE)])
        pltpu.sync_copy(src_hbm.at[tid], src_v)
        pltpu.sync_copy(dst_hbm.at[tid], dst_v)
        plsc.subcore_barrier()

        def body(j, carry):
            pltpu.async_copy(table_hbm.at[src_v.at[j]], rows_v, sem).wait()
            pltpu.sync_copy(rows_v, acc.at[dst_v.at[j]], add=True)
            return carry

        lax.fori_loop(0, NCHUNK, body, 0)
        plsc.subcore_barrier()
        pltpu.sync_copy(acc.at[pl.ds(r0, ROWS_PER_TILE)],
                        out_hbm.at[pl.ds(cid * NP + r0, ROWS_PER_TILE)])

    return sc_agg


@functools.lru_cache(maxsize=None)
def _make_sc_deg():
    @functools.partial(
        pl.kernel,
        mesh=plsc.VectorSubcoreMesh(core_axis_name="c", subcore_axis_name="s"),
        out_type=jax.ShapeDtypeStruct((NC * NP, DEG_W), jnp.float32),
        scratch_types=[
            pltpu.VMEM((NCHUNK, CH), jnp.int32),
            pltpu.VMEM((CH, DEG_W), jnp.float32),
            pltpu.VMEM_SHARED((NP, DEG_W), jnp.float32),
            pltpu.SemaphoreType.DMA,
        ],
    )
    def sc_deg(dst_hbm, ones_hbm, zeros_hbm, out_hbm, dst_v, ones_v, acc,
               sem):
        cid = lax.axis_index("c")
        sid = lax.axis_index("s")
        tid = cid * NS + sid
        r0 = sid * ROWS_PER_TILE
        pltpu.sync_copy(zeros_hbm.at[pl.ds(r0, ROWS_PER_TILE)],
                        acc.at[pl.ds(r0, ROWS_PER_TILE)])
        pltpu.sync_copy(ones_hbm, ones_v)
        pltpu.sync_copy(dst_hbm.at[tid], dst_v)
        plsc.subcore_barrier()

        def body(j, carry):
            jg = jnp.minimum(j, NCHUNK - 1)

            @pl.when(j < NCHUNK)
            def _():
                pltpu.async_copy(ones_v, acc.at[dst_v.at[jg]], sem,
                                 add=True)

            @pl.when(j > 0)
            def _():
                pltpu.make_async_copy(
                    ones_v, acc.at[dst_v.at[jg]], sem).wait()

            return carry

        lax.fori_loop(0, NCHUNK + 1, body, 0)
        plsc.subcore_barrier()
        pltpu.sync_copy(acc.at[pl.ds(r0, ROWS_PER_TILE)],
                        out_hbm.at[pl.ds(cid * NP + r0, ROWS_PER_TILE)])

    return sc_deg


def _tc_emb_body(h_ref, w_ref, b_ref, o_ref):
    o_ref[...] = jnp.dot(h_ref[...], w_ref[...],
                         preferred_element_type=jnp.float32) + b_ref[...]


def _tc_layer_body(x_ref, s_ref, degp_ref, w_ref, b_ref, g_ref, bet_ref,
                   o_ref):
    x = x_ref[...]
    s = s_ref[0:N] + s_ref[NP:NP + N]
    dp = degp_ref[0:N] + degp_ref[NP:NP + N]
    inv = 1.0 / jnp.maximum(dp, 1.0)
    w = w_ref[...]
    bundle = (jnp.dot(x, w[0:D], preferred_element_type=jnp.float32)
              + jnp.dot(s * inv, w[D:2 * D],
                        preferred_element_type=jnp.float32)
              + b_ref[...])
    nrm = jnp.sqrt(jnp.sum(bundle * bundle, axis=1, keepdims=True))
    bundle = bundle / jnp.maximum(nrm, 1e-12)
    y = jnp.maximum(bundle, 0.0)
    mean = jnp.mean(y, axis=0, keepdims=True)
    var = jnp.mean((y - mean) ** 2, axis=0, keepdims=True)
    o_ref[...] = x + (y - mean) * lax.rsqrt(var + 1e-5) * g_ref[...] \
        + bet_ref[...]


@functools.lru_cache(maxsize=None)
def _make_tc():
    tc_emb = pl.pallas_call(
        _tc_emb_body,
        out_shape=jax.ShapeDtypeStruct((N, D), jnp.float32),
    )
    tc_layer = pl.pallas_call(
        _tc_layer_body,
        out_shape=jax.ShapeDtypeStruct((N, D), jnp.float32),
    )
    return tc_emb, tc_layer


def kernel(h, e, edge_index, W_emb, b_emb, Ws, bs, gammas, betas):
    pad = PER_TILE - E // NW
    src = edge_index[0].astype(jnp.int32).reshape(NW, E // NW)
    dst = edge_index[1].astype(jnp.int32).reshape(NW, E // NW)
    src = jnp.pad(src, ((0, 0), (0, pad))).reshape(NW, NCHUNK, CH)
    dst = jnp.pad(dst, ((0, 0), (0, pad)),
                  constant_values=N).reshape(NW, NCHUNK, CH)
    zeros_g = jnp.zeros((NP, DEG_W), jnp.float32)
    ones_g = jnp.ones((CH, DEG_W), jnp.float32)
    sc_agg_rows = _make_sc_agg(D)
    sc_deg = _make_sc_deg()
    tc_emb, tc_layer = _make_tc()
    degp = sc_deg(dst, ones_g, zeros_g)
    x = tc_emb(h, W_emb, b_emb[None, :])
    zeros_d = degp[0:NP] * 0.0
    for l in range(L):
        s = sc_agg_rows(src, dst, x, zeros_d)
        x = tc_layer(x, s, degp, Ws[l], bs[l][None, :],
                     gammas[l][None, :], betas[l][None, :])
    return x
